# baseline (speedup 1.0000x reference)
import functools

import jax
import jax.numpy as jnp
from jax import lax
from jax.experimental import pallas as pl
from jax.experimental.pallas import tpu as pltpu
from jax.experimental.pallas import tpu_sc as plsc

_NC = 2
_NS = 16
_NW = _NC * _NS
_LANES = 16
_CHUNK = 128
_MCHUNK = 128
_DEGW = 128
_R = 256


def _cdiv(a, b):
    return (a + b - 1) // b


def _sc_mesh():
    return plsc.VectorSubcoreMesh(core_axis_name="c", subcore_axis_name="s")


def _sc_degree(rows3, n_pad, width):
    nchunks = rows3.shape[1]
    stripe = n_pad // _NS
    sub = width // _LANES

    @functools.partial(
        pl.kernel,
        out_type=jax.ShapeDtypeStruct((_NC, n_pad, width), jnp.float32),
        mesh=_sc_mesh(),
        scratch_types=[
            pltpu.VMEM((nchunks, _CHUNK), jnp.int32),
            pltpu.VMEM((_CHUNK, width), jnp.float32),
            pltpu.VMEM_SHARED((n_pad, width), jnp.float32),
        ],
    )
    def deg_kernel(rows_hbm, out_hbm, idx_v, cbuf, acc_sh):
        c = lax.axis_index("c")
        s = lax.axis_index("s")
        w = s * _NC + c

        pltpu.sync_copy(rows_hbm.at[w], idx_v)

        def fill(val):
            v16 = jnp.full((_LANES,), val, jnp.float32)

            def fi(j, carry):
                for k in range(sub):
                    cbuf[j, pl.ds(k * _LANES, _LANES)] = v16
                return carry

            lax.fori_loop(0, _CHUNK, fi, 0)

        fill(0.0)
        for t in range(stripe // _CHUNK):
            pltpu.sync_copy(cbuf, acc_sh.at[pl.ds(s * stripe + t * _CHUNK, _CHUNK)])
        fill(1.0)
        plsc.subcore_barrier()

        def body(i, carry):
            pltpu.sync_copy(cbuf, acc_sh.at[idx_v.at[i]], add=True)
            return carry

        lax.fori_loop(0, nchunks, body, 0)
        plsc.subcore_barrier()
        pltpu.sync_copy(acc_sh.at[pl.ds(s * stripe, stripe)],
                        out_hbm.at[c].at[pl.ds(s * stripe, stripe)])

    return deg_kernel(rows3)


def _sc_gather_scatter(y, ric3):
    n_pad, feat = y.shape
    nchunks, chunk = ric3.shape[1], ric3.shape[3]
    stripe = n_pad // _NS
    assert nchunks % 2 == 0

    @functools.partial(
        pl.kernel,
        out_type=jax.ShapeDtypeStruct((_NC, n_pad, feat), jnp.float32),
        mesh=_sc_mesh(),
        scratch_types=[
            [pltpu.VMEM((2, chunk), jnp.int32) for _ in range(2)],
            [pltpu.VMEM((chunk, feat), jnp.float32) for _ in range(2)],
            [pltpu.SemaphoreType.DMA for _ in range(2)],
            [pltpu.SemaphoreType.DMA for _ in range(2)],
            pltpu.VMEM_SHARED((n_pad, feat), jnp.float32),
        ],
    )
    def msg_kernel(y_hbm, ric_hbm, out_hbm, ibuf, gbuf, semi, semg, acc_sh):
        c = lax.axis_index("c")
        s = lax.axis_index("s")
        w = s * _NC + c
        pltpu.sync_copy(y_hbm.at[pl.ds(s * stripe, stripe)],
                        acc_sh.at[pl.ds(s * stripe, stripe)])
        plsc.subcore_barrier()

        ric_w = ric_hbm.at[w]
        for b in range(2):
            pltpu.async_copy(ric_w.at[b], ibuf[b], semi[b])
        pltpu.make_async_copy(ric_w.at[0], ibuf[0], semi[0]).wait()
        pltpu.async_copy(y_hbm.at[ibuf[0].at[0]], gbuf[0], semg[0])

        def body(j, carry):
            k0 = 2 * j
            for b in range(2):
                k = k0 + b
                b1 = 1 - b
                pltpu.make_async_copy(ric_w.at[k], ibuf[b1], semi[b1]).wait()
                pltpu.async_copy(y_hbm.at[ibuf[b1].at[0]], gbuf[b1], semg[b1])
                pltpu.make_async_copy(y_hbm.at[ibuf[b].at[0]], gbuf[b],
                                      semg[b]).wait()
                pltpu.sync_copy(gbuf[b], acc_sh.at[ibuf[b].at[1]], add=True)
                nxt = jnp.minimum(k + 2, nchunks - 1)
                pltpu.async_copy(ric_w.at[nxt], ibuf[b], semi[b])
            return carry

        lax.fori_loop(0, nchunks // 2, body, 0)
        pltpu.make_async_copy(ric_w.at[0], ibuf[1], semi[1]).wait()
        pltpu.make_async_copy(y_hbm.at[ibuf[0].at[0]], gbuf[0], semg[0]).wait()
        plsc.subcore_barrier()
        pltpu.sync_copy(acc_sh.at[pl.ds(s * stripe, stripe)],
                        out_hbm.at[c].at[pl.ds(s * stripe, stripe)])

    return msg_kernel(y, ric3)


def _tc_lin1(x_pad, d0, d1, wt1, b1r, n_real):
    n_pad, fin = x_pad.shape
    hdim = wt1.shape[1]

    def body(x_ref, d0_ref, d1_ref, w_ref, b_ref, y_ref, dis_ref):
        i = pl.program_id(0)
        deg = d0_ref[:, 0:1] + d1_ref[:, 0:1] + 1.0
        row = i * _R + lax.broadcasted_iota(jnp.int32, (_R, 1), 0)
        dis = jnp.where(row < n_real, lax.rsqrt(deg), 0.0)
        xl = jnp.dot(x_ref[...], w_ref[...],
                     preferred_element_type=jnp.float32) + b_ref[...]
        y_ref[...] = dis * xl
        dis_ref[...] = dis

    return pl.pallas_call(
        body,
        grid=(n_pad // _R,),
        in_specs=[
            pl.BlockSpec((_R, fin), lambda i: (i, 0)),
            pl.BlockSpec((_R, _DEGW), lambda i: (i, 0)),
            pl.BlockSpec((_R, _DEGW), lambda i: (i, 0)),
            pl.BlockSpec((fin, hdim), lambda i: (0, 0)),
            pl.BlockSpec((1, hdim), lambda i: (0, 0)),
        ],
        out_specs=[
            pl.BlockSpec((_R, hdim), lambda i: (i, 0)),
            pl.BlockSpec((_R, 1), lambda i: (i, 0)),
        ],
        out_shape=[
            jax.ShapeDtypeStruct((n_pad, hdim), jnp.float32),
            jax.ShapeDtypeStruct((n_pad, 1), jnp.float32),
        ],
    )(x_pad, d0, d1, wt1, b1r)


def _tc_lin2(a0, a1, y1, dis, wt2p, b2r):
    n_pad, hdim = y1.shape
    cpad = wt2p.shape[1]

    def body(a0_ref, a1_ref, y_ref, dis_ref, w_ref, b_ref, o_ref):
        dis = dis_ref[...]
        hid = jnp.maximum(dis * (a0_ref[...] + a1_ref[...] - y_ref[...]), 0.0)
        o_ref[...] = dis * (jnp.dot(hid, w_ref[...],
                                    preferred_element_type=jnp.float32)
                            + b_ref[...])

    return pl.pallas_call(
        body,
        grid=(n_pad // _R,),
        in_specs=[
            pl.BlockSpec((_R, hdim), lambda i: (i, 0)),
            pl.BlockSpec((_R, hdim), lambda i: (i, 0)),
            pl.BlockSpec((_R, hdim), lambda i: (i, 0)),
            pl.BlockSpec((_R, 1), lambda i: (i, 0)),
            pl.BlockSpec((hdim, cpad), lambda i: (0, 0)),
            pl.BlockSpec((1, cpad), lambda i: (0, 0)),
        ],
        out_specs=pl.BlockSpec((_R, cpad), lambda i: (i, 0)),
        out_shape=jax.ShapeDtypeStruct((n_pad, cpad), jnp.float32),
    )(a0, a1, y1, dis, wt2p, b2r)


def _tc_out(a0, a1, y2, dis, ncls):
    n_pad, cpad = y2.shape

    def body(a0_ref, a1_ref, y_ref, dis_ref, o_ref):
        z = dis_ref[...] * (a0_ref[...] + a1_ref[...] - y_ref[...])
        colmask = lax.broadcasted_iota(jnp.int32, (_R, cpad), 1) < ncls
        zm = jnp.where(colmask, z, -jnp.inf)
        m = jnp.max(zm, axis=1, keepdims=True)
        ez = jnp.where(colmask, jnp.exp(z - m), 0.0)
        lse = m + jnp.log(jnp.sum(ez, axis=1, keepdims=True))
        o_ref[...] = (z - lse)[:, :ncls]

    return pl.pallas_call(
        body,
        grid=(n_pad // _R,),
        in_specs=[
            pl.BlockSpec((_R, cpad), lambda i: (i, 0)),
            pl.BlockSpec((_R, cpad), lambda i: (i, 0)),
            pl.BlockSpec((_R, cpad), lambda i: (i, 0)),
            pl.BlockSpec((_R, 1), lambda i: (i, 0)),
        ],
        out_specs=pl.BlockSpec((_R, ncls), lambda i: (i, 0)),
        out_shape=jax.ShapeDtypeStruct((n_pad, ncls), jnp.float32),
    )(a0, a1, y2, dis)


def kernel(x, edge_index, owned_nodes, num_nodes, W1, b1, W2, b2):
    n, fin = x.shape
    hdim = W1.shape[0]
    ncls = W2.shape[0]
    e = edge_index.shape[1]

    n_pad = _cdiv(n + 1, _R) * _R
    cpad = _cdiv(ncls, 128) * 128

    rows = edge_index[0].astype(jnp.int32)
    cols = edge_index[1].astype(jnp.int32)

    def _pad3(v, chunk, mult):
        nc = _cdiv(e, _NW * chunk)
        nc = _cdiv(nc, mult) * mult
        ep = _NW * nc * chunk
        fill = n + jnp.arange(ep - e, dtype=jnp.int32) % (n_pad - n)
        return jnp.concatenate([v, fill]).reshape(_NW, nc, chunk)

    rows3d = _pad3(rows, _CHUNK, 1)
    ric3 = jnp.stack([_pad3(rows, _MCHUNK, 2), _pad3(cols, _MCHUNK, 2)], axis=2)
    x_pad = jnp.pad(x, ((0, n_pad - n), (0, 0)))
    wt1 = W1.T
    b1r = b1.reshape(1, hdim)
    wt2p = jnp.pad(W2, ((0, cpad - ncls), (0, 0))).T
    b2r = jnp.pad(b2, (0, cpad - ncls)).reshape(1, cpad)

    deg = _sc_degree(rows3d, n_pad, _DEGW)
    y1, dis = _tc_lin1(x_pad, deg[0], deg[1], wt1, b1r, n)
    acc1 = _sc_gather_scatter(y1, ric3)
    y2 = _tc_lin2(acc1[0], acc1[1], y1, dis, wt2p, b2r)
    acc2 = _sc_gather_scatter(y2, ric3)
    outp = _tc_out(acc2[0], acc2[1], y2, dis, ncls)
    return outp[:n]


# speedup vs baseline: 1.0757x; 1.0757x over previous
import functools

import jax
import jax.numpy as jnp
from jax import lax
from jax.experimental import pallas as pl
from jax.experimental.pallas import tpu as pltpu
from jax.experimental.pallas import tpu_sc as plsc

_NC = 2
_NS = 16
_NW = _NC * _NS
_LANES = 16
_CHUNK = 128
_MCHUNK = 112
_DEGW = 128
_R = 256


def _cdiv(a, b):
    return (a + b - 1) // b


def _sc_mesh():
    return plsc.VectorSubcoreMesh(core_axis_name="c", subcore_axis_name="s")


def _sc_degree(rows3, n_pad, width):
    nchunks = rows3.shape[1]
    stripe = n_pad // _NS
    sub = width // _LANES

    @functools.partial(
        pl.kernel,
        out_type=jax.ShapeDtypeStruct((_NC, n_pad, width), jnp.float32),
        mesh=_sc_mesh(),
        scratch_types=[
            pltpu.VMEM((nchunks, _CHUNK), jnp.int32),
            pltpu.VMEM((_CHUNK, width), jnp.float32),
            pltpu.VMEM_SHARED((n_pad, width), jnp.float32),
        ],
    )
    def deg_kernel(rows_hbm, out_hbm, idx_v, cbuf, acc_sh):
        c = lax.axis_index("c")
        s = lax.axis_index("s")
        w = s * _NC + c

        pltpu.sync_copy(rows_hbm.at[w], idx_v)

        def fill(val):
            v16 = jnp.full((_LANES,), val, jnp.float32)

            def fi(j, carry):
                for k in range(sub):
                    cbuf[j, pl.ds(k * _LANES, _LANES)] = v16
                return carry

            lax.fori_loop(0, _CHUNK, fi, 0)

        fill(0.0)
        for t in range(stripe // _CHUNK):
            pltpu.sync_copy(cbuf, acc_sh.at[pl.ds(s * stripe + t * _CHUNK, _CHUNK)])
        fill(1.0)
        plsc.subcore_barrier()

        def body(i, carry):
            pltpu.sync_copy(cbuf, acc_sh.at[idx_v.at[i]], add=True)
            return carry

        lax.fori_loop(0, nchunks, body, 0)
        plsc.subcore_barrier()
        pltpu.sync_copy(acc_sh.at[pl.ds(s * stripe, stripe)],
                        out_hbm.at[c].at[pl.ds(s * stripe, stripe)])

    return deg_kernel(rows3)


def _sc_gather_scatter(y, ric3):
    n_pad, feat = y.shape
    nchunks, chunk = ric3.shape[1], ric3.shape[3]
    stripe = n_pad // _NS
    assert nchunks % 3 == 0

    @functools.partial(
        pl.kernel,
        out_type=jax.ShapeDtypeStruct((_NC, n_pad, feat), jnp.float32),
        mesh=_sc_mesh(),
        scratch_types=[
            [pltpu.VMEM((2, chunk), jnp.int32) for _ in range(3)],
            [pltpu.VMEM((chunk, feat), jnp.float32) for _ in range(3)],
            [pltpu.SemaphoreType.DMA for _ in range(3)],
            [pltpu.SemaphoreType.DMA for _ in range(3)],
            pltpu.VMEM_SHARED((n_pad, feat), jnp.float32),
        ],
    )
    def msg_kernel(y_hbm, ric_hbm, out_hbm, ibuf, gbuf, semi, semg, acc_sh):
        c = lax.axis_index("c")
        s = lax.axis_index("s")
        w = s * _NC + c
        pltpu.sync_copy(y_hbm.at[pl.ds(s * stripe, stripe)],
                        acc_sh.at[pl.ds(s * stripe, stripe)])
        plsc.subcore_barrier()

        ric_w = ric_hbm.at[w]
        for b in range(3):
            pltpu.async_copy(ric_w.at[b], ibuf[b], semi[b])
        pltpu.make_async_copy(ric_w.at[0], ibuf[0], semi[0]).wait()
        pltpu.async_copy(y_hbm.at[ibuf[0].at[0]], gbuf[0], semg[0])

        def body(j, carry):
            k0 = 3 * j
            for b in range(3):
                k = k0 + b
                b1 = (b + 1) % 3
                pltpu.make_async_copy(ric_w.at[k], ibuf[b1], semi[b1]).wait()
                pltpu.async_copy(y_hbm.at[ibuf[b1].at[0]], gbuf[b1], semg[b1])
                pltpu.make_async_copy(y_hbm.at[ibuf[b].at[0]], gbuf[b],
                                      semg[b]).wait()
                pltpu.sync_copy(gbuf[b], acc_sh.at[ibuf[b].at[1]], add=True)
                nxt = jnp.minimum(k + 3, nchunks - 1)
                pltpu.async_copy(ric_w.at[nxt], ibuf[b], semi[b])
            return carry

        lax.fori_loop(0, nchunks // 3, body, 0)
        pltpu.make_async_copy(ric_w.at[0], ibuf[1], semi[1]).wait()
        pltpu.make_async_copy(ric_w.at[0], ibuf[2], semi[2]).wait()
        pltpu.make_async_copy(y_hbm.at[ibuf[0].at[0]], gbuf[0], semg[0]).wait()
        plsc.subcore_barrier()
        pltpu.sync_copy(acc_sh.at[pl.ds(s * stripe, stripe)],
                        out_hbm.at[c].at[pl.ds(s * stripe, stripe)])

    return msg_kernel(y, ric3)


def _tc_lin1(x_pad, d0, d1, wt1, b1r, n_real):
    n_pad, fin = x_pad.shape
    hdim = wt1.shape[1]

    def body(x_ref, d0_ref, d1_ref, w_ref, b_ref, y_ref, dis_ref):
        i = pl.program_id(0)
        deg = d0_ref[:, 0:1] + d1_ref[:, 0:1] + 1.0
        row = i * _R + lax.broadcasted_iota(jnp.int32, (_R, 1), 0)
        dis = jnp.where(row < n_real, lax.rsqrt(deg), 0.0)
        xl = jnp.dot(x_ref[...], w_ref[...],
                     preferred_element_type=jnp.float32) + b_ref[...]
        y_ref[...] = dis * xl
        dis_ref[...] = dis

    return pl.pallas_call(
        body,
        grid=(n_pad // _R,),
        in_specs=[
            pl.BlockSpec((_R, fin), lambda i: (i, 0)),
            pl.BlockSpec((_R, _DEGW), lambda i: (i, 0)),
            pl.BlockSpec((_R, _DEGW), lambda i: (i, 0)),
            pl.BlockSpec((fin, hdim), lambda i: (0, 0)),
            pl.BlockSpec((1, hdim), lambda i: (0, 0)),
        ],
        out_specs=[
            pl.BlockSpec((_R, hdim), lambda i: (i, 0)),
            pl.BlockSpec((_R, 1), lambda i: (i, 0)),
        ],
        out_shape=[
            jax.ShapeDtypeStruct((n_pad, hdim), jnp.float32),
            jax.ShapeDtypeStruct((n_pad, 1), jnp.float32),
        ],
    )(x_pad, d0, d1, wt1, b1r)


def _tc_lin2(a0, a1, y1, dis, wt2p, b2r):
    n_pad, hdim = y1.shape
    cpad = wt2p.shape[1]

    def body(a0_ref, a1_ref, y_ref, dis_ref, w_ref, b_ref, o_ref):
        dis = dis_ref[...]
        hid = jnp.maximum(dis * (a0_ref[...] + a1_ref[...] - y_ref[...]), 0.0)
        o_ref[...] = dis * (jnp.dot(hid, w_ref[...],
                                    preferred_element_type=jnp.float32)
                            + b_ref[...])

    return pl.pallas_call(
        body,
        grid=(n_pad // _R,),
        in_specs=[
            pl.BlockSpec((_R, hdim), lambda i: (i, 0)),
            pl.BlockSpec((_R, hdim), lambda i: (i, 0)),
            pl.BlockSpec((_R, hdim), lambda i: (i, 0)),
            pl.BlockSpec((_R, 1), lambda i: (i, 0)),
            pl.BlockSpec((hdim, cpad), lambda i: (0, 0)),
            pl.BlockSpec((1, cpad), lambda i: (0, 0)),
        ],
        out_specs=pl.BlockSpec((_R, cpad), lambda i: (i, 0)),
        out_shape=jax.ShapeDtypeStruct((n_pad, cpad), jnp.float32),
    )(a0, a1, y1, dis, wt2p, b2r)


def _tc_out(a0, a1, y2, dis, ncls):
    n_pad, cpad = y2.shape

    def body(a0_ref, a1_ref, y_ref, dis_ref, o_ref):
        z = dis_ref[...] * (a0_ref[...] + a1_ref[...] - y_ref[...])
        colmask = lax.broadcasted_iota(jnp.int32, (_R, cpad), 1) < ncls
        zm = jnp.where(colmask, z, -jnp.inf)
        m = jnp.max(zm, axis=1, keepdims=True)
        ez = jnp.where(colmask, jnp.exp(z - m), 0.0)
        lse = m + jnp.log(jnp.sum(ez, axis=1, keepdims=True))
        o_ref[...] = (z - lse)[:, :ncls]

    return pl.pallas_call(
        body,
        grid=(n_pad // _R,),
        in_specs=[
            pl.BlockSpec((_R, cpad), lambda i: (i, 0)),
            pl.BlockSpec((_R, cpad), lambda i: (i, 0)),
            pl.BlockSpec((_R, cpad), lambda i: (i, 0)),
            pl.BlockSpec((_R, 1), lambda i: (i, 0)),
        ],
        out_specs=pl.BlockSpec((_R, ncls), lambda i: (i, 0)),
        out_shape=jax.ShapeDtypeStruct((n_pad, ncls), jnp.float32),
    )(a0, a1, y2, dis)


def kernel(x, edge_index, owned_nodes, num_nodes, W1, b1, W2, b2):
    n, fin = x.shape
    hdim = W1.shape[0]
    ncls = W2.shape[0]
    e = edge_index.shape[1]

    n_pad = _cdiv(n + 1, _R) * _R
    cpad = _cdiv(ncls, 128) * 128

    rows = edge_index[0].astype(jnp.int32)
    cols = edge_index[1].astype(jnp.int32)

    def _pad3(v, chunk, mult):
        nc = _cdiv(e, _NW * chunk)
        nc = _cdiv(nc, mult) * mult
        ep = _NW * nc * chunk
        fill = n + jnp.arange(ep - e, dtype=jnp.int32) % (n_pad - n)
        return jnp.concatenate([v, fill]).reshape(_NW, nc, chunk)

    rows3d = _pad3(rows, _CHUNK, 1)
    ric3 = jnp.stack([_pad3(rows, _MCHUNK, 3), _pad3(cols, _MCHUNK, 3)], axis=2)
    x_pad = jnp.pad(x, ((0, n_pad - n), (0, 0)))
    wt1 = W1.T
    b1r = b1.reshape(1, hdim)
    wt2p = jnp.pad(W2, ((0, cpad - ncls), (0, 0))).T
    b2r = jnp.pad(b2, (0, cpad - ncls)).reshape(1, cpad)

    deg = _sc_degree(rows3d, n_pad, _DEGW)
    y1, dis = _tc_lin1(x_pad, deg[0], deg[1], wt1, b1r, n)
    acc1 = _sc_gather_scatter(y1, ric3)
    y2 = _tc_lin2(acc1[0], acc1[1], y1, dis, wt2p, b2r)
    acc2 = _sc_gather_scatter(y2, ric3)
    outp = _tc_out(acc2[0], acc2[1], y2, dis, ncls)
    return outp[:n]


# speedup vs baseline: 1.0777x; 1.0018x over previous
import functools

import jax
import jax.numpy as jnp
from jax import lax
from jax.experimental import pallas as pl
from jax.experimental.pallas import tpu as pltpu
from jax.experimental.pallas import tpu_sc as plsc

_NC = 2
_NS = 16
_NW = _NC * _NS
_LANES = 16
_CHUNK = 128
_MCHUNK = 112
_DEGW = 128
_R = 256


def _cdiv(a, b):
    return (a + b - 1) // b


def _sc_mesh():
    return plsc.VectorSubcoreMesh(core_axis_name="c", subcore_axis_name="s")


def _sc_degree(rows3, n_pad, width):
    nchunks = rows3.shape[1]
    stripe = n_pad // _NS
    sub = width // _LANES

    @functools.partial(
        pl.kernel,
        out_type=jax.ShapeDtypeStruct((_NC, n_pad, width), jnp.float32),
        mesh=_sc_mesh(),
        scratch_types=[
            pltpu.VMEM((nchunks, _CHUNK), jnp.int32),
            pltpu.VMEM((_CHUNK, width), jnp.float32),
            pltpu.VMEM_SHARED((n_pad, width), jnp.float32),
        ],
    )
    def deg_kernel(rows_hbm, out_hbm, idx_v, cbuf, acc_sh):
        c = lax.axis_index("c")
        s = lax.axis_index("s")
        w = s * _NC + c

        pltpu.sync_copy(rows_hbm.at[w], idx_v)

        def fill(val):
            v16 = jnp.full((_LANES,), val, jnp.float32)

            def fi(j, carry):
                for k in range(sub):
                    cbuf[j, pl.ds(k * _LANES, _LANES)] = v16
                return carry

            lax.fori_loop(0, _CHUNK, fi, 0)

        fill(0.0)
        for t in range(stripe // _CHUNK):
            pltpu.sync_copy(cbuf, acc_sh.at[pl.ds(s * stripe + t * _CHUNK, _CHUNK)])
        fill(1.0)
        plsc.subcore_barrier()

        def body(i, carry):
            pltpu.sync_copy(cbuf, acc_sh.at[idx_v.at[i]], add=True)
            return carry

        lax.fori_loop(0, nchunks, body, 0)
        plsc.subcore_barrier()
        pltpu.sync_copy(acc_sh.at[pl.ds(s * stripe, stripe)],
                        out_hbm.at[c].at[pl.ds(s * stripe, stripe)])

    return deg_kernel(rows3)


def _sc_gather_scatter(y, ric3):
    n_pad, feat = y.shape
    nchunks, chunk = ric3.shape[1], ric3.shape[3]
    stripe = n_pad // _NS
    assert nchunks % 3 == 0

    @functools.partial(
        pl.kernel,
        out_type=jax.ShapeDtypeStruct((_NC, n_pad, feat), jnp.float32),
        mesh=_sc_mesh(),
        scratch_types=[
            [pltpu.VMEM((2, chunk), jnp.int32) for _ in range(3)],
            [pltpu.VMEM((chunk, feat), jnp.float32) for _ in range(3)],
            [pltpu.SemaphoreType.DMA for _ in range(3)],
            [pltpu.SemaphoreType.DMA for _ in range(3)],
            pltpu.VMEM_SHARED((n_pad, feat), jnp.float32),
        ],
    )
    def msg_kernel(y_hbm, ric_hbm, out_hbm, ibuf, gbuf, semi, semg, acc_sh):
        c = lax.axis_index("c")
        s = lax.axis_index("s")
        w = s * _NC + c
        pltpu.sync_copy(y_hbm.at[pl.ds(s * stripe, stripe)],
                        acc_sh.at[pl.ds(s * stripe, stripe)])
        plsc.subcore_barrier()

        ric_w = ric_hbm.at[w]
        for b in range(3):
            pltpu.async_copy(ric_w.at[b], ibuf[b], semi[b])
        pltpu.make_async_copy(ric_w.at[0], ibuf[0], semi[0]).wait()
        pltpu.async_copy(y_hbm.at[ibuf[0].at[0]], gbuf[0], semg[0])

        def body(j, carry):
            k0 = 3 * j
            for b in range(3):
                k = k0 + b
                b1 = (b + 1) % 3
                pltpu.make_async_copy(ric_w.at[k], ibuf[b1], semi[b1]).wait()
                pltpu.async_copy(y_hbm.at[ibuf[b1].at[0]], gbuf[b1], semg[b1])
                pltpu.make_async_copy(y_hbm.at[ibuf[b].at[0]], gbuf[b],
                                      semg[b]).wait()
                pltpu.sync_copy(gbuf[b], acc_sh.at[ibuf[b].at[1]], add=True)
                nxt = jnp.minimum(k + 3, nchunks - 1)
                pltpu.async_copy(ric_w.at[nxt], ibuf[b], semi[b])
            return carry

        lax.fori_loop(0, nchunks // 3, body, 0)
        pltpu.make_async_copy(ric_w.at[0], ibuf[1], semi[1]).wait()
        pltpu.make_async_copy(ric_w.at[0], ibuf[2], semi[2]).wait()
        pltpu.make_async_copy(y_hbm.at[ibuf[0].at[0]], gbuf[0], semg[0]).wait()
        plsc.subcore_barrier()
        pltpu.sync_copy(acc_sh.at[pl.ds(s * stripe, stripe)],
                        out_hbm.at[c].at[pl.ds(s * stripe, stripe)])

    return msg_kernel(y, ric3)


def _tc_mm1(x_pad, wt1, b1r):
    n_pad, fin = x_pad.shape
    hdim = wt1.shape[1]

    def body(x_ref, w_ref, b_ref, u_ref):
        u_ref[...] = jnp.dot(x_ref[...], w_ref[...],
                             preferred_element_type=jnp.float32) + b_ref[...]

    return pl.pallas_call(
        body,
        grid=(n_pad // _R,),
        in_specs=[
            pl.BlockSpec((_R, fin), lambda i: (i, 0)),
            pl.BlockSpec((fin, hdim), lambda i: (0, 0)),
            pl.BlockSpec((1, hdim), lambda i: (0, 0)),
        ],
        out_specs=pl.BlockSpec((_R, hdim), lambda i: (i, 0)),
        out_shape=jax.ShapeDtypeStruct((n_pad, hdim), jnp.float32),
    )(x_pad, wt1, b1r)


def _tc_scale1(u1, d0, d1, n_real):
    n_pad, hdim = u1.shape

    def body(u_ref, d0_ref, d1_ref, y_ref, dis_ref):
        i = pl.program_id(0)
        deg = d0_ref[:, 0:1] + d1_ref[:, 0:1] + 1.0
        row = i * _R + lax.broadcasted_iota(jnp.int32, (_R, 1), 0)
        dis = jnp.where(row < n_real, lax.rsqrt(deg), 0.0)
        y_ref[...] = dis * u_ref[...]
        dis_ref[...] = dis

    return pl.pallas_call(
        body,
        grid=(n_pad // _R,),
        in_specs=[
            pl.BlockSpec((_R, hdim), lambda i: (i, 0)),
            pl.BlockSpec((_R, _DEGW), lambda i: (i, 0)),
            pl.BlockSpec((_R, _DEGW), lambda i: (i, 0)),
        ],
        out_specs=[
            pl.BlockSpec((_R, hdim), lambda i: (i, 0)),
            pl.BlockSpec((_R, 1), lambda i: (i, 0)),
        ],
        out_shape=[
            jax.ShapeDtypeStruct((n_pad, hdim), jnp.float32),
            jax.ShapeDtypeStruct((n_pad, 1), jnp.float32),
        ],
    )(u1, d0, d1)


def _tc_lin2(a0, a1, y1, dis, wt2p, b2r):
    n_pad, hdim = y1.shape
    cpad = wt2p.shape[1]

    def body(a0_ref, a1_ref, y_ref, dis_ref, w_ref, b_ref, o_ref):
        dis = dis_ref[...]
        hid = jnp.maximum(dis * (a0_ref[...] + a1_ref[...] - y_ref[...]), 0.0)
        o_ref[...] = dis * (jnp.dot(hid, w_ref[...],
                                    preferred_element_type=jnp.float32)
                            + b_ref[...])

    return pl.pallas_call(
        body,
        grid=(n_pad // _R,),
        in_specs=[
            pl.BlockSpec((_R, hdim), lambda i: (i, 0)),
            pl.BlockSpec((_R, hdim), lambda i: (i, 0)),
            pl.BlockSpec((_R, hdim), lambda i: (i, 0)),
            pl.BlockSpec((_R, 1), lambda i: (i, 0)),
            pl.BlockSpec((hdim, cpad), lambda i: (0, 0)),
            pl.BlockSpec((1, cpad), lambda i: (0, 0)),
        ],
        out_specs=pl.BlockSpec((_R, cpad), lambda i: (i, 0)),
        out_shape=jax.ShapeDtypeStruct((n_pad, cpad), jnp.float32),
    )(a0, a1, y1, dis, wt2p, b2r)


def _tc_out(a0, a1, y2, dis, ncls):
    n_pad, cpad = y2.shape

    def body(a0_ref, a1_ref, y_ref, dis_ref, o_ref):
        z = dis_ref[...] * (a0_ref[...] + a1_ref[...] - y_ref[...])
        colmask = lax.broadcasted_iota(jnp.int32, (_R, cpad), 1) < ncls
        zm = jnp.where(colmask, z, -jnp.inf)
        m = jnp.max(zm, axis=1, keepdims=True)
        ez = jnp.where(colmask, jnp.exp(z - m), 0.0)
        lse = m + jnp.log(jnp.sum(ez, axis=1, keepdims=True))
        o_ref[...] = (z - lse)[:, :ncls]

    return pl.pallas_call(
        body,
        grid=(n_pad // _R,),
        in_specs=[
            pl.BlockSpec((_R, cpad), lambda i: (i, 0)),
            pl.BlockSpec((_R, cpad), lambda i: (i, 0)),
            pl.BlockSpec((_R, cpad), lambda i: (i, 0)),
            pl.BlockSpec((_R, 1), lambda i: (i, 0)),
        ],
        out_specs=pl.BlockSpec((_R, ncls), lambda i: (i, 0)),
        out_shape=jax.ShapeDtypeStruct((n_pad, ncls), jnp.float32),
    )(a0, a1, y2, dis)


def kernel(x, edge_index, owned_nodes, num_nodes, W1, b1, W2, b2):
    n, fin = x.shape
    hdim = W1.shape[0]
    ncls = W2.shape[0]
    e = edge_index.shape[1]

    n_pad = _cdiv(n + 1, _R) * _R
    cpad = _cdiv(ncls, 128) * 128

    rows = edge_index[0].astype(jnp.int32)
    cols = edge_index[1].astype(jnp.int32)

    def _pad3(v, chunk, mult):
        nc = _cdiv(e, _NW * chunk)
        nc = _cdiv(nc, mult) * mult
        ep = _NW * nc * chunk
        fill = n + jnp.arange(ep - e, dtype=jnp.int32) % (n_pad - n)
        return jnp.concatenate([v, fill]).reshape(_NW, nc, chunk)

    rows3d = _pad3(rows, _CHUNK, 1)
    ric3 = jnp.stack([_pad3(rows, _MCHUNK, 3), _pad3(cols, _MCHUNK, 3)], axis=2)
    x_pad = jnp.pad(x, ((0, n_pad - n), (0, 0)))
    wt1 = W1.T
    b1r = b1.reshape(1, hdim)
    wt2p = jnp.pad(W2, ((0, cpad - ncls), (0, 0))).T
    b2r = jnp.pad(b2, (0, cpad - ncls)).reshape(1, cpad)

    deg = _sc_degree(rows3d, n_pad, _DEGW)
    u1 = _tc_mm1(x_pad, wt1, b1r)
    y1, dis = _tc_scale1(u1, deg[0], deg[1], n)
    acc1 = _sc_gather_scatter(y1, ric3)
    y2 = _tc_lin2(acc1[0], acc1[1], y1, dis, wt2p, b2r)
    acc2 = _sc_gather_scatter(y2, ric3)
    outp = _tc_out(acc2[0], acc2[1], y2, dis, ncls)
    return outp[:n]


# speedup vs baseline: 1.2121x; 1.1247x over previous
import functools

import jax
import jax.numpy as jnp
from jax import lax
from jax.experimental import pallas as pl
from jax.experimental.pallas import tpu as pltpu
from jax.experimental.pallas import tpu_sc as plsc

_NC = 2
_NS = 16
_NW = _NC * _NS
_LANES = 16
_CHUNK = 128
_MCHUNK = 112
_DEGW = 128
_R = 1024


def _cdiv(a, b):
    return (a + b - 1) // b


def _sc_mesh():
    return plsc.VectorSubcoreMesh(core_axis_name="c", subcore_axis_name="s")


def _sc_degree(rows3, n_pad, width):
    nchunks = rows3.shape[1]
    stripe = n_pad // _NS
    sub = width // _LANES

    @functools.partial(
        pl.kernel,
        out_type=jax.ShapeDtypeStruct((_NC, n_pad, width), jnp.float32),
        mesh=_sc_mesh(),
        scratch_types=[
            pltpu.VMEM((nchunks, _CHUNK), jnp.int32),
            pltpu.VMEM((_CHUNK, width), jnp.float32),
            pltpu.VMEM_SHARED((n_pad, width), jnp.float32),
        ],
    )
    def deg_kernel(rows_hbm, out_hbm, idx_v, cbuf, acc_sh):
        c = lax.axis_index("c")
        s = lax.axis_index("s")
        w = s * _NC + c

        pltpu.sync_copy(rows_hbm.at[w], idx_v)

        def fill(val):
            v16 = jnp.full((_LANES,), val, jnp.float32)

            def fi(j, carry):
                for k in range(sub):
                    cbuf[j, pl.ds(k * _LANES, _LANES)] = v16
                return carry

            lax.fori_loop(0, _CHUNK, fi, 0)

        fill(0.0)
        for t in range(stripe // _CHUNK):
            pltpu.sync_copy(cbuf, acc_sh.at[pl.ds(s * stripe + t * _CHUNK, _CHUNK)])
        fill(1.0)
        plsc.subcore_barrier()

        def body(i, carry):
            pltpu.sync_copy(cbuf, acc_sh.at[idx_v.at[i]], add=True)
            return carry

        lax.fori_loop(0, nchunks, body, 0)
        plsc.subcore_barrier()
        pltpu.sync_copy(acc_sh.at[pl.ds(s * stripe, stripe)],
                        out_hbm.at[c].at[pl.ds(s * stripe, stripe)])

    return deg_kernel(rows3)


def _sc_gather_scatter(y, ric3):
    n_pad, feat = y.shape
    nchunks, chunk = ric3.shape[1], ric3.shape[3]
    stripe = n_pad // _NS
    assert nchunks % 3 == 0

    @functools.partial(
        pl.kernel,
        out_type=jax.ShapeDtypeStruct((_NC, n_pad, feat), jnp.float32),
        mesh=_sc_mesh(),
        scratch_types=[
            [pltpu.VMEM((2, chunk), jnp.int32) for _ in range(3)],
            [pltpu.VMEM((chunk, feat), jnp.float32) for _ in range(3)],
            [pltpu.SemaphoreType.DMA for _ in range(3)],
            [pltpu.SemaphoreType.DMA for _ in range(3)],
            pltpu.VMEM_SHARED((n_pad, feat), jnp.float32),
        ],
    )
    def msg_kernel(y_hbm, ric_hbm, out_hbm, ibuf, gbuf, semi, semg, acc_sh):
        c = lax.axis_index("c")
        s = lax.axis_index("s")
        w = s * _NC + c
        pltpu.sync_copy(y_hbm.at[pl.ds(s * stripe, stripe)],
                        acc_sh.at[pl.ds(s * stripe, stripe)])
        plsc.subcore_barrier()

        ric_w = ric_hbm.at[w]
        for b in range(3):
            pltpu.async_copy(ric_w.at[b], ibuf[b], semi[b])
        pltpu.make_async_copy(ric_w.at[0], ibuf[0], semi[0]).wait()
        pltpu.async_copy(y_hbm.at[ibuf[0].at[0]], gbuf[0], semg[0])

        def body(j, carry):
            k0 = 3 * j
            for b in range(3):
                k = k0 + b
                b1 = (b + 1) % 3
                pltpu.make_async_copy(ric_w.at[k], ibuf[b1], semi[b1]).wait()
                pltpu.async_copy(y_hbm.at[ibuf[b1].at[0]], gbuf[b1], semg[b1])
                pltpu.make_async_copy(y_hbm.at[ibuf[b].at[0]], gbuf[b],
                                      semg[b]).wait()
                pltpu.sync_copy(gbuf[b], acc_sh.at[ibuf[b].at[1]], add=True)
                nxt = jnp.minimum(k + 3, nchunks - 1)
                pltpu.async_copy(ric_w.at[nxt], ibuf[b], semi[b])
            return carry

        lax.fori_loop(0, nchunks // 3, body, 0)
        pltpu.make_async_copy(ric_w.at[0], ibuf[1], semi[1]).wait()
        pltpu.make_async_copy(ric_w.at[0], ibuf[2], semi[2]).wait()
        pltpu.make_async_copy(y_hbm.at[ibuf[0].at[0]], gbuf[0], semg[0]).wait()
        plsc.subcore_barrier()
        pltpu.sync_copy(acc_sh.at[pl.ds(s * stripe, stripe)],
                        out_hbm.at[c].at[pl.ds(s * stripe, stripe)])

    return msg_kernel(y, ric3)


def _tc_mm1(x_pad, wt1, b1r):
    n_pad, fin = x_pad.shape
    hdim = wt1.shape[1]

    def body(x_ref, w_ref, b_ref, u_ref):
        u_ref[...] = jnp.dot(x_ref[...], w_ref[...],
                             preferred_element_type=jnp.float32) + b_ref[...]

    return pl.pallas_call(
        body,
        grid=(n_pad // _R,),
        in_specs=[
            pl.BlockSpec((_R, fin), lambda i: (i, 0)),
            pl.BlockSpec((fin, hdim), lambda i: (0, 0)),
            pl.BlockSpec((1, hdim), lambda i: (0, 0)),
        ],
        out_specs=pl.BlockSpec((_R, hdim), lambda i: (i, 0)),
        out_shape=jax.ShapeDtypeStruct((n_pad, hdim), jnp.float32),
    )(x_pad, wt1, b1r)


def _tc_scale1(u1, d0, d1, n_real):
    n_pad, hdim = u1.shape

    def body(u_ref, d0_ref, d1_ref, y_ref, dis_ref):
        i = pl.program_id(0)
        deg = d0_ref[:, 0:1] + d1_ref[:, 0:1] + 1.0
        row = i * _R + lax.broadcasted_iota(jnp.int32, (_R, 1), 0)
        dis = jnp.where(row < n_real, lax.rsqrt(deg), 0.0)
        y_ref[...] = dis * u_ref[...]
        dis_ref[...] = dis

    return pl.pallas_call(
        body,
        grid=(n_pad // _R,),
        in_specs=[
            pl.BlockSpec((_R, hdim), lambda i: (i, 0)),
            pl.BlockSpec((_R, _DEGW), lambda i: (i, 0)),
            pl.BlockSpec((_R, _DEGW), lambda i: (i, 0)),
        ],
        out_specs=[
            pl.BlockSpec((_R, hdim), lambda i: (i, 0)),
            pl.BlockSpec((_R, 1), lambda i: (i, 0)),
        ],
        out_shape=[
            jax.ShapeDtypeStruct((n_pad, hdim), jnp.float32),
            jax.ShapeDtypeStruct((n_pad, 1), jnp.float32),
        ],
    )(u1, d0, d1)


def _tc_lin2(a0, a1, y1, dis, wt2p, b2r):
    n_pad, hdim = y1.shape
    cpad = wt2p.shape[1]

    def body(a0_ref, a1_ref, y_ref, dis_ref, w_ref, b_ref, o_ref):
        dis = dis_ref[...]
        hid = jnp.maximum(dis * (a0_ref[...] + a1_ref[...] - y_ref[...]), 0.0)
        o_ref[...] = dis * (jnp.dot(hid, w_ref[...],
                                    preferred_element_type=jnp.float32)
                            + b_ref[...])

    return pl.pallas_call(
        body,
        grid=(n_pad // _R,),
        in_specs=[
            pl.BlockSpec((_R, hdim), lambda i: (i, 0)),
            pl.BlockSpec((_R, hdim), lambda i: (i, 0)),
            pl.BlockSpec((_R, hdim), lambda i: (i, 0)),
            pl.BlockSpec((_R, 1), lambda i: (i, 0)),
            pl.BlockSpec((hdim, cpad), lambda i: (0, 0)),
            pl.BlockSpec((1, cpad), lambda i: (0, 0)),
        ],
        out_specs=pl.BlockSpec((_R, cpad), lambda i: (i, 0)),
        out_shape=jax.ShapeDtypeStruct((n_pad, cpad), jnp.float32),
    )(a0, a1, y1, dis, wt2p, b2r)


def _tc_out(a0, a1, y2, dis, ncls):
    n_pad, cpad = y2.shape

    def body(a0_ref, a1_ref, y_ref, dis_ref, o_ref):
        z = dis_ref[...] * (a0_ref[...] + a1_ref[...] - y_ref[...])
        colmask = lax.broadcasted_iota(jnp.int32, (_R, cpad), 1) < ncls
        zm = jnp.where(colmask, z, -jnp.inf)
        m = jnp.max(zm, axis=1, keepdims=True)
        ez = jnp.where(colmask, jnp.exp(z - m), 0.0)
        lse = m + jnp.log(jnp.sum(ez, axis=1, keepdims=True))
        o_ref[...] = (z - lse)[:, :ncls]

    return pl.pallas_call(
        body,
        grid=(n_pad // _R,),
        in_specs=[
            pl.BlockSpec((_R, cpad), lambda i: (i, 0)),
            pl.BlockSpec((_R, cpad), lambda i: (i, 0)),
            pl.BlockSpec((_R, cpad), lambda i: (i, 0)),
            pl.BlockSpec((_R, 1), lambda i: (i, 0)),
        ],
        out_specs=pl.BlockSpec((_R, ncls), lambda i: (i, 0)),
        out_shape=jax.ShapeDtypeStruct((n_pad, ncls), jnp.float32),
    )(a0, a1, y2, dis)


def kernel(x, edge_index, owned_nodes, num_nodes, W1, b1, W2, b2):
    n, fin = x.shape
    hdim = W1.shape[0]
    ncls = W2.shape[0]
    e = edge_index.shape[1]

    n_pad = _cdiv(n + 1, _R) * _R
    cpad = _cdiv(ncls, 128) * 128

    rows = edge_index[0].astype(jnp.int32)
    cols = edge_index[1].astype(jnp.int32)

    def _pad3(v, chunk, mult):
        nc = _cdiv(e, _NW * chunk)
        nc = _cdiv(nc, mult) * mult
        ep = _NW * nc * chunk
        fill = n + jnp.arange(ep - e, dtype=jnp.int32) % (n_pad - n)
        return jnp.concatenate([v, fill]).reshape(_NW, nc, chunk)

    rows3d = _pad3(rows, _CHUNK, 1)
    ric3 = jnp.stack([_pad3(rows, _MCHUNK, 3), _pad3(cols, _MCHUNK, 3)], axis=2)
    x_pad = jnp.pad(x, ((0, n_pad - n), (0, 0)))
    wt1 = W1.T
    b1r = b1.reshape(1, hdim)
    wt2p = jnp.pad(W2, ((0, cpad - ncls), (0, 0))).T
    b2r = jnp.pad(b2, (0, cpad - ncls)).reshape(1, cpad)

    deg = _sc_degree(rows3d, n_pad, _DEGW)
    u1 = _tc_mm1(x_pad, wt1, b1r)
    y1, dis = _tc_scale1(u1, deg[0], deg[1], n)
    acc1 = _sc_gather_scatter(y1, ric3)
    y2 = _tc_lin2(acc1[0], acc1[1], y1, dis, wt2p, b2r)
    acc2 = _sc_gather_scatter(y2, ric3)
    outp = _tc_out(acc2[0], acc2[1], y2, dis, ncls)
    return outp[:n]


# speedup vs baseline: 1.2330x; 1.0172x over previous
import functools

import jax
import jax.numpy as jnp
from jax import lax
from jax.experimental import pallas as pl
from jax.experimental.pallas import tpu as pltpu
from jax.experimental.pallas import tpu_sc as plsc

_NC = 2
_NS = 16
_NW = _NC * _NS
_LANES = 16
_CHUNK = 128
_MCHUNK = 112
_DEGW = 128
_R = 2048


def _cdiv(a, b):
    return (a + b - 1) // b


def _sc_mesh():
    return plsc.VectorSubcoreMesh(core_axis_name="c", subcore_axis_name="s")


def _sc_degree(rows3, n_pad, width):
    nchunks = rows3.shape[1]
    stripe = n_pad // _NS
    sub = width // _LANES

    @functools.partial(
        pl.kernel,
        out_type=jax.ShapeDtypeStruct((_NC, n_pad, width), jnp.float32),
        mesh=_sc_mesh(),
        scratch_types=[
            pltpu.VMEM((nchunks, _CHUNK), jnp.int32),
            pltpu.VMEM((_CHUNK, width), jnp.float32),
            pltpu.VMEM_SHARED((n_pad, width), jnp.float32),
        ],
    )
    def deg_kernel(rows_hbm, out_hbm, idx_v, cbuf, acc_sh):
        c = lax.axis_index("c")
        s = lax.axis_index("s")
        w = s * _NC + c

        pltpu.sync_copy(rows_hbm.at[w], idx_v)

        def fill(val):
            v16 = jnp.full((_LANES,), val, jnp.float32)

            def fi(j, carry):
                for k in range(sub):
                    cbuf[j, pl.ds(k * _LANES, _LANES)] = v16
                return carry

            lax.fori_loop(0, _CHUNK, fi, 0)

        fill(0.0)
        for t in range(stripe // _CHUNK):
            pltpu.sync_copy(cbuf, acc_sh.at[pl.ds(s * stripe + t * _CHUNK, _CHUNK)])
        fill(1.0)
        plsc.subcore_barrier()

        def body(i, carry):
            pltpu.sync_copy(cbuf, acc_sh.at[idx_v.at[i]], add=True)
            return carry

        lax.fori_loop(0, nchunks, body, 0)
        plsc.subcore_barrier()
        pltpu.sync_copy(acc_sh.at[pl.ds(s * stripe, stripe)],
                        out_hbm.at[c].at[pl.ds(s * stripe, stripe)])

    return deg_kernel(rows3)


def _sc_gather_scatter(y, ric3):
    n_pad, feat = y.shape
    nchunks, chunk = ric3.shape[1], ric3.shape[3]
    stripe = n_pad // _NS
    assert nchunks % 3 == 0

    @functools.partial(
        pl.kernel,
        out_type=jax.ShapeDtypeStruct((_NC, n_pad, feat), jnp.float32),
        mesh=_sc_mesh(),
        scratch_types=[
            [pltpu.VMEM((2, chunk), jnp.int32) for _ in range(3)],
            [pltpu.VMEM((chunk, feat), jnp.float32) for _ in range(3)],
            [pltpu.SemaphoreType.DMA for _ in range(3)],
            [pltpu.SemaphoreType.DMA for _ in range(3)],
            pltpu.VMEM_SHARED((n_pad, feat), jnp.float32),
        ],
    )
    def msg_kernel(y_hbm, ric_hbm, out_hbm, ibuf, gbuf, semi, semg, acc_sh):
        c = lax.axis_index("c")
        s = lax.axis_index("s")
        w = s * _NC + c
        pltpu.sync_copy(y_hbm.at[pl.ds(s * stripe, stripe)],
                        acc_sh.at[pl.ds(s * stripe, stripe)])
        plsc.subcore_barrier()

        ric_w = ric_hbm.at[w]
        for b in range(3):
            pltpu.async_copy(ric_w.at[b], ibuf[b], semi[b])
        pltpu.make_async_copy(ric_w.at[0], ibuf[0], semi[0]).wait()
        pltpu.async_copy(y_hbm.at[ibuf[0].at[0]], gbuf[0], semg[0])

        def body(j, carry):
            k0 = 3 * j
            for b in range(3):
                k = k0 + b
                b1 = (b + 1) % 3
                pltpu.make_async_copy(ric_w.at[k], ibuf[b1], semi[b1]).wait()
                pltpu.async_copy(y_hbm.at[ibuf[b1].at[0]], gbuf[b1], semg[b1])
                pltpu.make_async_copy(y_hbm.at[ibuf[b].at[0]], gbuf[b],
                                      semg[b]).wait()
                pltpu.sync_copy(gbuf[b], acc_sh.at[ibuf[b].at[1]], add=True)
                nxt = jnp.minimum(k + 3, nchunks - 1)
                pltpu.async_copy(ric_w.at[nxt], ibuf[b], semi[b])
            return carry

        lax.fori_loop(0, nchunks // 3, body, 0)
        pltpu.make_async_copy(ric_w.at[0], ibuf[1], semi[1]).wait()
        pltpu.make_async_copy(ric_w.at[0], ibuf[2], semi[2]).wait()
        pltpu.make_async_copy(y_hbm.at[ibuf[0].at[0]], gbuf[0], semg[0]).wait()
        plsc.subcore_barrier()
        pltpu.sync_copy(acc_sh.at[pl.ds(s * stripe, stripe)],
                        out_hbm.at[c].at[pl.ds(s * stripe, stripe)])

    return msg_kernel(y, ric3)


def _tc_mm1(x_pad, wt1, b1r):
    n_pad, fin = x_pad.shape
    hdim = wt1.shape[1]

    def body(x_ref, w_ref, b_ref, u_ref):
        u_ref[...] = jnp.dot(x_ref[...], w_ref[...],
                             preferred_element_type=jnp.float32) + b_ref[...]

    return pl.pallas_call(
        body,
        grid=(n_pad // _R,),
        in_specs=[
            pl.BlockSpec((_R, fin), lambda i: (i, 0)),
            pl.BlockSpec((fin, hdim), lambda i: (0, 0)),
            pl.BlockSpec((1, hdim), lambda i: (0, 0)),
        ],
        out_specs=pl.BlockSpec((_R, hdim), lambda i: (i, 0)),
        out_shape=jax.ShapeDtypeStruct((n_pad, hdim), jnp.float32),
    )(x_pad, wt1, b1r)


def _tc_scale1(u1, d0, d1, n_real):
    n_pad, hdim = u1.shape

    def body(u_ref, d0_ref, d1_ref, y_ref, dis_ref):
        i = pl.program_id(0)
        deg = d0_ref[:, 0:1] + d1_ref[:, 0:1] + 1.0
        row = i * _R + lax.broadcasted_iota(jnp.int32, (_R, 1), 0)
        dis = jnp.where(row < n_real, lax.rsqrt(deg), 0.0)
        y_ref[...] = dis * u_ref[...]
        dis_ref[...] = dis

    return pl.pallas_call(
        body,
        grid=(n_pad // _R,),
        in_specs=[
            pl.BlockSpec((_R, hdim), lambda i: (i, 0)),
            pl.BlockSpec((_R, _DEGW), lambda i: (i, 0)),
            pl.BlockSpec((_R, _DEGW), lambda i: (i, 0)),
        ],
        out_specs=[
            pl.BlockSpec((_R, hdim), lambda i: (i, 0)),
            pl.BlockSpec((_R, 1), lambda i: (i, 0)),
        ],
        out_shape=[
            jax.ShapeDtypeStruct((n_pad, hdim), jnp.float32),
            jax.ShapeDtypeStruct((n_pad, 1), jnp.float32),
        ],
    )(u1, d0, d1)


def _tc_lin2(a0, a1, y1, dis, wt2p, b2r):
    n_pad, hdim = y1.shape
    cpad = wt2p.shape[1]

    def body(a0_ref, a1_ref, y_ref, dis_ref, w_ref, b_ref, o_ref):
        dis = dis_ref[...]
        hid = jnp.maximum(dis * (a0_ref[...] + a1_ref[...] - y_ref[...]), 0.0)
        o_ref[...] = dis * (jnp.dot(hid, w_ref[...],
                                    preferred_element_type=jnp.float32)
                            + b_ref[...])

    return pl.pallas_call(
        body,
        grid=(n_pad // _R,),
        in_specs=[
            pl.BlockSpec((_R, hdim), lambda i: (i, 0)),
            pl.BlockSpec((_R, hdim), lambda i: (i, 0)),
            pl.BlockSpec((_R, hdim), lambda i: (i, 0)),
            pl.BlockSpec((_R, 1), lambda i: (i, 0)),
            pl.BlockSpec((hdim, cpad), lambda i: (0, 0)),
            pl.BlockSpec((1, cpad), lambda i: (0, 0)),
        ],
        out_specs=pl.BlockSpec((_R, cpad), lambda i: (i, 0)),
        out_shape=jax.ShapeDtypeStruct((n_pad, cpad), jnp.float32),
    )(a0, a1, y1, dis, wt2p, b2r)


def _tc_out(a0, a1, y2, dis, ncls):
    n_pad, cpad = y2.shape

    def body(a0_ref, a1_ref, y_ref, dis_ref, o_ref):
        z = dis_ref[...] * (a0_ref[...] + a1_ref[...] - y_ref[...])
        colmask = lax.broadcasted_iota(jnp.int32, (_R, cpad), 1) < ncls
        zm = jnp.where(colmask, z, -jnp.inf)
        m = jnp.max(zm, axis=1, keepdims=True)
        ez = jnp.where(colmask, jnp.exp(z - m), 0.0)
        lse = m + jnp.log(jnp.sum(ez, axis=1, keepdims=True))
        o_ref[...] = (z - lse)[:, :ncls]

    return pl.pallas_call(
        body,
        grid=(n_pad // _R,),
        in_specs=[
            pl.BlockSpec((_R, cpad), lambda i: (i, 0)),
            pl.BlockSpec((_R, cpad), lambda i: (i, 0)),
            pl.BlockSpec((_R, cpad), lambda i: (i, 0)),
            pl.BlockSpec((_R, 1), lambda i: (i, 0)),
        ],
        out_specs=pl.BlockSpec((_R, ncls), lambda i: (i, 0)),
        out_shape=jax.ShapeDtypeStruct((n_pad, ncls), jnp.float32),
    )(a0, a1, y2, dis)


def kernel(x, edge_index, owned_nodes, num_nodes, W1, b1, W2, b2):
    n, fin = x.shape
    hdim = W1.shape[0]
    ncls = W2.shape[0]
    e = edge_index.shape[1]

    n_pad = _cdiv(n + 1, _R) * _R
    cpad = _cdiv(ncls, 128) * 128

    rows = edge_index[0].astype(jnp.int32)
    cols = edge_index[1].astype(jnp.int32)

    def _pad3(v, chunk, mult):
        nc = _cdiv(e, _NW * chunk)
        nc = _cdiv(nc, mult) * mult
        ep = _NW * nc * chunk
        fill = n + jnp.arange(ep - e, dtype=jnp.int32) % (n_pad - n)
        return jnp.concatenate([v, fill]).reshape(_NW, nc, chunk)

    rows3d = _pad3(rows, _CHUNK, 1)
    ric3 = jnp.stack([_pad3(rows, _MCHUNK, 3), _pad3(cols, _MCHUNK, 3)], axis=2)
    x_pad = jnp.pad(x, ((0, n_pad - n), (0, 0)))
    wt1 = W1.T
    b1r = b1.reshape(1, hdim)
    wt2p = jnp.pad(W2, ((0, cpad - ncls), (0, 0))).T
    b2r = jnp.pad(b2, (0, cpad - ncls)).reshape(1, cpad)

    deg = _sc_degree(rows3d, n_pad, _DEGW)
    u1 = _tc_mm1(x_pad, wt1, b1r)
    y1, dis = _tc_scale1(u1, deg[0], deg[1], n)
    acc1 = _sc_gather_scatter(y1, ric3)
    y2 = _tc_lin2(acc1[0], acc1[1], y1, dis, wt2p, b2r)
    acc2 = _sc_gather_scatter(y2, ric3)
    outp = _tc_out(acc2[0], acc2[1], y2, dis, ncls)
    return outp[:n]
